# Initial kernel scaffold; baseline (speedup 1.0000x reference)
import jax
import jax.numpy as jnp
from jax.experimental import pallas as pl


def kernel(x, w_h, b_h, w_s, w_o, b_o):
    raise NotImplementedError("write your pallas kernel here")


# speedup vs baseline: 1.4752x; 1.4752x over previous
import functools

import jax
import jax.numpy as jnp
from jax import lax
from jax.experimental import pallas as pl
from jax.experimental.pallas import tpu as pltpu

_K = 4
_BIG = 1e30


def _proj_kernel(x_ref, wh_ref, bh_ref, ws_ref, wo3_ref, bo_ref,
                 h_ref, s_ref, xo_ref):
    xv = x_ref[...]
    h_ref[...] = (jnp.dot(xv, wh_ref[...], preferred_element_type=jnp.float32)
                  + bh_ref[...])
    s_ref[...] = jnp.dot(xv, ws_ref[...], preferred_element_type=jnp.float32)
    xo_ref[...] = (jnp.dot(xv, wo3_ref[...], preferred_element_type=jnp.float32)
                   + bo_ref[...])


def _knn_kernel(tile, n_total, sq_ref, s_all_ref, nbr_ref, w_ref):
    t0 = pl.program_id(0) * tile
    T = sq_ref.shape[0]
    N = s_all_ref.shape[0]

    s_q = sq_ref[...]
    s_all = s_all_ref[...]
    qn = jnp.sum(s_q * s_q, axis=-1, keepdims=True)
    an = lax.dot_general(jnp.ones((1, s_q.shape[1]), jnp.float32),
                         s_all * s_all,
                         dimension_numbers=(((1,), (1,)), ((), ())),
                         preferred_element_type=jnp.float32)
    cross = lax.dot_general(s_q, s_all,
                            dimension_numbers=(((1,), (1,)), ((), ())),
                            preferred_element_type=jnp.float32)
    d2 = qn + an - 2.0 * cross

    big = jnp.float32(_BIG)
    row_ids = t0 + lax.broadcasted_iota(jnp.int32, (T, N), 0)
    col_ids = lax.broadcasted_iota(jnp.int32, (T, N), 1)
    d2 = jnp.where(col_ids == row_ids, big, d2)

    idx_cols = []
    w_cols = []
    for r in range(_K):
        m = jnp.min(d2, axis=-1, keepdims=True)
        cand = jnp.where(d2 <= m, col_ids, n_total)
        j = jnp.min(cand, axis=-1, keepdims=True)
        idx_cols.append(j)
        w_cols.append(jnp.exp(-m))
        if r < _K - 1:
            d2 = jnp.where(cand == j, big, d2)
    nbr_ref[...] = jnp.concatenate(idx_cols, axis=-1)
    w_ref[...] = jnp.concatenate(w_cols, axis=-1)


def _agg_kernel(tile, nbr_smem, xo_ref, w_ref, h3_ref, wo12_ref, out_ref,
                g0, g1, g2, g3):
    P = h3_ref.shape[2]
    base = pl.program_id(0) * (tile * _K)
    for mi in range(tile):
        o = base + mi * _K
        g0[mi] = h3_ref[nbr_smem[o], 0]
        g1[mi] = h3_ref[nbr_smem[o + 1], 0]
        g2[mi] = h3_ref[nbr_smem[o + 2], 0]
        g3[mi] = h3_ref[nbr_smem[o + 3], 0]
    wv = w_ref[...]
    m0 = g0[...] * wv[:, 0:1]
    m1 = g1[...] * wv[:, 1:2]
    m2 = g2[...] * wv[:, 2:3]
    m3 = g3[...] * wv[:, 3:4]
    mean = (((m0 + m1) + m2) + m3) * jnp.float32(1.0 / _K)
    mx = jnp.maximum(jnp.maximum(jnp.maximum(m0, m1), m2), m3)
    out_ref[...] = (jnp.dot(mean, wo12_ref[0:P, :],
                            preferred_element_type=jnp.float32)
                    + jnp.dot(mx, wo12_ref[P:2 * P, :],
                              preferred_element_type=jnp.float32)
                    + xo_ref[...])


def kernel(x, w_h, b_h, w_s, w_o, b_o):
    N, C_in = x.shape
    P = w_h.shape[1]
    S = w_s.shape[1]
    C_out = w_o.shape[1]

    x = x.astype(jnp.float32)
    w_h = w_h.astype(jnp.float32)
    w_s = w_s.astype(jnp.float32)
    w_o = w_o.astype(jnp.float32)
    bh2 = b_h.reshape(1, P).astype(jnp.float32)
    bo2 = b_o.reshape(1, C_out).astype(jnp.float32)
    wo3 = w_o[2 * P:2 * P + C_in, :]
    wo12 = w_o[0:2 * P, :]

    t1 = min(64, N)
    nt1 = N // t1
    h_l, s_l, xo = pl.pallas_call(
        _proj_kernel,
        out_shape=(jax.ShapeDtypeStruct((N, P), jnp.float32),
                   jax.ShapeDtypeStruct((N, S), jnp.float32),
                   jax.ShapeDtypeStruct((N, C_out), jnp.float32)),
        grid=(nt1,),
        in_specs=[pl.BlockSpec((t1, C_in), lambda i: (i, 0)),
                  pl.BlockSpec((C_in, P), lambda i: (0, 0)),
                  pl.BlockSpec((1, P), lambda i: (0, 0)),
                  pl.BlockSpec((C_in, S), lambda i: (0, 0)),
                  pl.BlockSpec((C_in, C_out), lambda i: (0, 0)),
                  pl.BlockSpec((1, C_out), lambda i: (0, 0))],
        out_specs=(pl.BlockSpec((t1, P), lambda i: (i, 0)),
                   pl.BlockSpec((t1, S), lambda i: (i, 0)),
                   pl.BlockSpec((t1, C_out), lambda i: (i, 0))),
        compiler_params=pltpu.CompilerParams(
            dimension_semantics=("parallel",)),
    )(x, w_h, bh2, w_s, wo3, bo2)

    t2 = min(128, N)
    nt2 = N // t2
    nbr, wgt = pl.pallas_call(
        functools.partial(_knn_kernel, t2, N),
        out_shape=(jax.ShapeDtypeStruct((N, _K), jnp.int32),
                   jax.ShapeDtypeStruct((N, _K), jnp.float32)),
        grid=(nt2,),
        in_specs=[pl.BlockSpec((t2, S), lambda i: (i, 0)),
                  pl.BlockSpec((N, S), lambda i: (0, 0))],
        out_specs=(pl.BlockSpec((t2, _K), lambda i: (i, 0)),
                   pl.BlockSpec((t2, _K), lambda i: (i, 0))),
        compiler_params=pltpu.CompilerParams(
            dimension_semantics=("parallel",),
            vmem_limit_bytes=64 * 1024 * 1024),
    )(s_l, s_l)

    t3 = min(128, N)
    nt3 = N // t3
    h3 = h_l.reshape(N, 1, P)
    nbr_flat = nbr.reshape(N * _K)
    out = pl.pallas_call(
        functools.partial(_agg_kernel, t3),
        out_shape=jax.ShapeDtypeStruct((N, C_out), jnp.float32),
        grid_spec=pltpu.PrefetchScalarGridSpec(
            num_scalar_prefetch=1,
            grid=(nt3,),
            in_specs=[pl.BlockSpec((t3, C_out), lambda i, nb: (i, 0)),
                      pl.BlockSpec((t3, _K), lambda i, nb: (i, 0)),
                      pl.BlockSpec((N, 1, P), lambda i, nb: (0, 0, 0)),
                      pl.BlockSpec((2 * P, C_out), lambda i, nb: (0, 0))],
            out_specs=pl.BlockSpec((t3, C_out), lambda i, nb: (i, 0)),
            scratch_shapes=[pltpu.VMEM((t3, P), jnp.float32)
                            for _ in range(_K)],
        ),
        compiler_params=pltpu.CompilerParams(
            dimension_semantics=("arbitrary",)),
    )(nbr_flat, xo, wgt, h3, wo12)

    return out, nbr, s_l


# speedup vs baseline: 2.0160x; 1.3666x over previous
import functools

import jax
import jax.numpy as jnp
from jax import lax
from jax.experimental import pallas as pl
from jax.experimental.pallas import tpu as pltpu

_K = 4
_BIG = 1e30


def _proj_kernel(x_ref, wh_ref, bh_ref, ws_ref, wo3_ref, bo_ref,
                 h_ref, s_ref, xo_ref):
    xv = x_ref[...]
    h_ref[...] = (jnp.dot(xv, wh_ref[...], preferred_element_type=jnp.float32)
                  + bh_ref[...])
    s_ref[...] = jnp.dot(xv, ws_ref[...], preferred_element_type=jnp.float32)
    xo_ref[...] = (jnp.dot(xv, wo3_ref[...], preferred_element_type=jnp.float32)
                   + bo_ref[...])


def _an_kernel(s_all_ref, an_ref):
    s_all = s_all_ref[...]
    an_ref[...] = lax.dot_general(
        jnp.ones((1, s_all.shape[1]), jnp.float32), s_all * s_all,
        dimension_numbers=(((1,), (1,)), ((), ())),
        preferred_element_type=jnp.float32)


def _knn_kernel(tile, n_total, sq_ref, s_all_ref, an_ref, nbr_ref, w_ref):
    t0 = pl.program_id(0) * tile
    T = tile
    N = s_all_ref.shape[0]
    NB = N // 128

    s_q = sq_ref[...]
    qn = jnp.sum(s_q * s_q, axis=-1, keepdims=True)
    cross = lax.dot_general(s_q, s_all_ref[...],
                            dimension_numbers=(((1,), (1,)), ((), ())),
                            preferred_element_type=jnp.float32)
    anv = an_ref[...]

    big = jnp.float32(_BIG)
    rowf = (t0 + lax.broadcasted_iota(jnp.int32, (T, 1), 0)
            ).astype(jnp.float32)
    lane = lax.broadcasted_iota(jnp.int32, (T, 128), 1).astype(
        jnp.float32)

    r1 = jnp.full((T, 128), big, jnp.float32)
    r2 = r1
    r3 = r1
    r4 = r1
    b1 = jnp.zeros((T, 128), jnp.float32)
    b2 = b1
    b3 = b1
    b4 = b1
    for blk in range(NB):
        sl = slice(blk * 128, (blk + 1) * 128)
        v = (qn + anv[:, sl]) - 2.0 * cross[:, sl]
        colb = lane + jnp.float32(blk * 128)
        v = jnp.where(colb == rowf, big, v)
        fb = jnp.float32(blk)
        c1 = v < r1
        c2 = v < r2
        c3 = v < r3
        c4 = v < r4
        r4 = jnp.where(c3, r3, jnp.where(c4, v, r4))
        b4 = jnp.where(c3, b3, jnp.where(c4, fb, b4))
        r3 = jnp.where(c2, r2, jnp.where(c3, v, r3))
        b3 = jnp.where(c2, b2, jnp.where(c3, fb, b3))
        r2 = jnp.where(c1, r1, jnp.where(c2, v, r2))
        b2 = jnp.where(c1, b1, jnp.where(c2, fb, b2))
        r1 = jnp.where(c1, v, r1)
        b1 = jnp.where(c1, fb, b1)

    V = jnp.concatenate([r1, r2, r3, r4], axis=-1)
    lane4 = jnp.concatenate([lane, lane, lane, lane], axis=-1)
    C = (jnp.concatenate([b1, b2, b3, b4], axis=-1) * jnp.float32(128.0)
         + lane4)

    nf = jnp.float32(n_total)
    idx_cols = []
    w_cols = []
    for rr in range(_K):
        m = jnp.min(V, axis=-1, keepdims=True)
        cand = jnp.where(V <= m, C, nf)
        j = jnp.min(cand, axis=-1, keepdims=True)
        idx_cols.append(j)
        w_cols.append(jnp.exp(-m))
        if rr < _K - 1:
            V = jnp.where(cand == j, big, V)
    nbr_ref[...] = jnp.concatenate(idx_cols, axis=-1).astype(jnp.int32)
    w_ref[...] = jnp.concatenate(w_cols, axis=-1)


def _agg_kernel(tile, nbr_smem, xo_ref, w_ref, h3_ref, wo12_ref, out_ref,
                g0, g1, g2, g3):
    P = h3_ref.shape[2]
    base = pl.program_id(0) * (tile * _K)
    for mi in range(tile):
        o = base + mi * _K
        g0[mi] = h3_ref[nbr_smem[o], 0]
        g1[mi] = h3_ref[nbr_smem[o + 1], 0]
        g2[mi] = h3_ref[nbr_smem[o + 2], 0]
        g3[mi] = h3_ref[nbr_smem[o + 3], 0]
    wv = w_ref[...]
    m0 = g0[...] * wv[:, 0:1]
    m1 = g1[...] * wv[:, 1:2]
    m2 = g2[...] * wv[:, 2:3]
    m3 = g3[...] * wv[:, 3:4]
    mean = (((m0 + m1) + m2) + m3) * jnp.float32(1.0 / _K)
    mx = jnp.maximum(jnp.maximum(jnp.maximum(m0, m1), m2), m3)
    out_ref[...] = (jnp.dot(mean, wo12_ref[0:P, :],
                            preferred_element_type=jnp.float32)
                    + jnp.dot(mx, wo12_ref[P:2 * P, :],
                              preferred_element_type=jnp.float32)
                    + xo_ref[...])


def kernel(x, w_h, b_h, w_s, w_o, b_o):
    N, C_in = x.shape
    P = w_h.shape[1]
    S = w_s.shape[1]
    C_out = w_o.shape[1]

    x = x.astype(jnp.float32)
    w_h = w_h.astype(jnp.float32)
    w_s = w_s.astype(jnp.float32)
    w_o = w_o.astype(jnp.float32)
    bh2 = b_h.reshape(1, P).astype(jnp.float32)
    bo2 = b_o.reshape(1, C_out).astype(jnp.float32)
    wo3 = w_o[2 * P:2 * P + C_in, :]
    wo12 = w_o[0:2 * P, :]

    t1 = min(64, N)
    nt1 = N // t1
    h_l, s_l, xo = pl.pallas_call(
        _proj_kernel,
        out_shape=(jax.ShapeDtypeStruct((N, P), jnp.float32),
                   jax.ShapeDtypeStruct((N, S), jnp.float32),
                   jax.ShapeDtypeStruct((N, C_out), jnp.float32)),
        grid=(nt1,),
        in_specs=[pl.BlockSpec((t1, C_in), lambda i: (i, 0)),
                  pl.BlockSpec((C_in, P), lambda i: (0, 0)),
                  pl.BlockSpec((1, P), lambda i: (0, 0)),
                  pl.BlockSpec((C_in, S), lambda i: (0, 0)),
                  pl.BlockSpec((C_in, C_out), lambda i: (0, 0)),
                  pl.BlockSpec((1, C_out), lambda i: (0, 0))],
        out_specs=(pl.BlockSpec((t1, P), lambda i: (i, 0)),
                   pl.BlockSpec((t1, S), lambda i: (i, 0)),
                   pl.BlockSpec((t1, C_out), lambda i: (i, 0))),
        compiler_params=pltpu.CompilerParams(
            dimension_semantics=("parallel",)),
    )(x, w_h, bh2, w_s, wo3, bo2)

    an = pl.pallas_call(
        _an_kernel,
        out_shape=jax.ShapeDtypeStruct((1, N), jnp.float32),
        grid=(1,),
        in_specs=[pl.BlockSpec((N, S), lambda i: (0, 0))],
        out_specs=pl.BlockSpec((1, N), lambda i: (0, 0)),
        compiler_params=pltpu.CompilerParams(
            dimension_semantics=("arbitrary",),
            vmem_limit_bytes=64 * 1024 * 1024),
    )(s_l)

    t2 = min(128, N)
    nt2 = N // t2
    nbr, wgt = pl.pallas_call(
        functools.partial(_knn_kernel, t2, N),
        out_shape=(jax.ShapeDtypeStruct((N, _K), jnp.int32),
                   jax.ShapeDtypeStruct((N, _K), jnp.float32)),
        grid=(nt2,),
        in_specs=[pl.BlockSpec((t2, S), lambda i: (i, 0)),
                  pl.BlockSpec((N, S), lambda i: (0, 0)),
                  pl.BlockSpec((1, N), lambda i: (0, 0))],
        out_specs=(pl.BlockSpec((t2, _K), lambda i: (i, 0)),
                   pl.BlockSpec((t2, _K), lambda i: (i, 0))),
        compiler_params=pltpu.CompilerParams(
            dimension_semantics=("parallel",),
            vmem_limit_bytes=64 * 1024 * 1024),
    )(s_l, s_l, an)

    t3 = min(128, N)
    nt3 = N // t3
    h3 = h_l.reshape(N, 1, P)
    nbr_flat = nbr.reshape(N * _K)
    out = pl.pallas_call(
        functools.partial(_agg_kernel, t3),
        out_shape=jax.ShapeDtypeStruct((N, C_out), jnp.float32),
        grid_spec=pltpu.PrefetchScalarGridSpec(
            num_scalar_prefetch=1,
            grid=(nt3,),
            in_specs=[pl.BlockSpec((t3, C_out), lambda i, nb: (i, 0)),
                      pl.BlockSpec((t3, _K), lambda i, nb: (i, 0)),
                      pl.BlockSpec((N, 1, P), lambda i, nb: (0, 0, 0)),
                      pl.BlockSpec((2 * P, C_out), lambda i, nb: (0, 0))],
            out_specs=pl.BlockSpec((t3, C_out), lambda i, nb: (i, 0)),
            scratch_shapes=[pltpu.VMEM((t3, P), jnp.float32)
                            for _ in range(_K)],
        ),
        compiler_params=pltpu.CompilerParams(
            dimension_semantics=("arbitrary",)),
    )(nbr_flat, xo, wgt, h3, wo12)

    return out, nbr, s_l


# speedup vs baseline: 2.0167x; 1.0003x over previous
import functools

import jax
import jax.numpy as jnp
from jax import lax
from jax.experimental import pallas as pl
from jax.experimental.pallas import tpu as pltpu

_K = 4
_BIG = 1e30


def _proj_kernel(x_ref, wh_ref, bh_ref, ws_ref, wo3_ref, bo_ref,
                 h_ref, s_ref, xo_ref):
    xv = x_ref[...]
    h_ref[...] = (jnp.dot(xv, wh_ref[...], preferred_element_type=jnp.float32)
                  + bh_ref[...])
    s_ref[...] = jnp.dot(xv, ws_ref[...], preferred_element_type=jnp.float32)
    xo_ref[...] = (jnp.dot(xv, wo3_ref[...], preferred_element_type=jnp.float32)
                   + bo_ref[...])


def _an_kernel(s_all_ref, an_ref):
    s_all = s_all_ref[...]
    an_ref[...] = lax.dot_general(
        jnp.ones((1, s_all.shape[1]), jnp.float32), s_all * s_all,
        dimension_numbers=(((1,), (1,)), ((), ())),
        preferred_element_type=jnp.float32)


def _knn_kernel(tile, n_total, sq_ref, s_all_ref, an_ref, nbr_ref, w_ref):
    t0 = pl.program_id(0) * tile
    T = tile
    N = s_all_ref.shape[0]
    NB = N // 128

    s_q = sq_ref[...]
    qn = jnp.sum(s_q * s_q, axis=-1, keepdims=True)
    anv = an_ref[...]

    big = jnp.float32(_BIG)
    rowf = (t0 + lax.broadcasted_iota(jnp.int32, (T, 1), 0)
            ).astype(jnp.float32)
    lane = lax.broadcasted_iota(jnp.int32, (T, 128), 1).astype(
        jnp.float32)

    r1 = jnp.full((T, 128), big, jnp.float32)
    r2 = r1
    r3 = r1
    r4 = r1
    b1 = jnp.zeros((T, 128), jnp.float32)
    b2 = b1
    b3 = b1
    b4 = b1
    CB = 16
    for blk in range(NB):
        if blk % CB == 0:
            ch = blk // CB
            cross_c = lax.dot_general(
                s_q, s_all_ref[ch * CB * 128:(ch + 1) * CB * 128, :],
                dimension_numbers=(((1,), (1,)), ((), ())),
                preferred_element_type=jnp.float32)
        sl = slice(blk * 128, (blk + 1) * 128)
        lsl = slice((blk % CB) * 128, (blk % CB + 1) * 128)
        v = (qn + anv[:, sl]) - 2.0 * cross_c[:, lsl]
        colb = lane + jnp.float32(blk * 128)
        v = jnp.where(colb == rowf, big, v)
        fb = jnp.float32(blk)
        c1 = v < r1
        c2 = v < r2
        c3 = v < r3
        c4 = v < r4
        r4 = jnp.where(c3, r3, jnp.where(c4, v, r4))
        b4 = jnp.where(c3, b3, jnp.where(c4, fb, b4))
        r3 = jnp.where(c2, r2, jnp.where(c3, v, r3))
        b3 = jnp.where(c2, b2, jnp.where(c3, fb, b3))
        r2 = jnp.where(c1, r1, jnp.where(c2, v, r2))
        b2 = jnp.where(c1, b1, jnp.where(c2, fb, b2))
        r1 = jnp.where(c1, v, r1)
        b1 = jnp.where(c1, fb, b1)

    V = jnp.concatenate([r1, r2, r3, r4], axis=-1)
    lane4 = jnp.concatenate([lane, lane, lane, lane], axis=-1)
    C = (jnp.concatenate([b1, b2, b3, b4], axis=-1) * jnp.float32(128.0)
         + lane4)

    nf = jnp.float32(n_total)
    idx_cols = []
    w_cols = []
    for rr in range(_K):
        m = jnp.min(V, axis=-1, keepdims=True)
        cand = jnp.where(V <= m, C, nf)
        j = jnp.min(cand, axis=-1, keepdims=True)
        idx_cols.append(j)
        w_cols.append(jnp.exp(-m))
        if rr < _K - 1:
            V = jnp.where(cand == j, big, V)
    nbr_ref[...] = jnp.concatenate(idx_cols, axis=-1).astype(jnp.int32)
    w_ref[...] = jnp.concatenate(w_cols, axis=-1)


def _agg_kernel(tile, nbr_smem, xo_ref, w_ref, h3_ref, wo12_ref, out_ref,
                g0, g1, g2, g3):
    P = h3_ref.shape[2]
    base = pl.program_id(0) * (tile * _K)
    for mi in range(tile):
        o = base + mi * _K
        g0[mi] = h3_ref[nbr_smem[o], 0]
        g1[mi] = h3_ref[nbr_smem[o + 1], 0]
        g2[mi] = h3_ref[nbr_smem[o + 2], 0]
        g3[mi] = h3_ref[nbr_smem[o + 3], 0]
    wv = w_ref[...]
    m0 = g0[...] * wv[:, 0:1]
    m1 = g1[...] * wv[:, 1:2]
    m2 = g2[...] * wv[:, 2:3]
    m3 = g3[...] * wv[:, 3:4]
    mean = (((m0 + m1) + m2) + m3) * jnp.float32(1.0 / _K)
    mx = jnp.maximum(jnp.maximum(jnp.maximum(m0, m1), m2), m3)
    out_ref[...] = (jnp.dot(mean, wo12_ref[0:P, :],
                            preferred_element_type=jnp.float32)
                    + jnp.dot(mx, wo12_ref[P:2 * P, :],
                              preferred_element_type=jnp.float32)
                    + xo_ref[...])


def kernel(x, w_h, b_h, w_s, w_o, b_o):
    N, C_in = x.shape
    P = w_h.shape[1]
    S = w_s.shape[1]
    C_out = w_o.shape[1]

    x = x.astype(jnp.float32)
    w_h = w_h.astype(jnp.float32)
    w_s = w_s.astype(jnp.float32)
    w_o = w_o.astype(jnp.float32)
    bh2 = b_h.reshape(1, P).astype(jnp.float32)
    bo2 = b_o.reshape(1, C_out).astype(jnp.float32)
    wo3 = w_o[2 * P:2 * P + C_in, :]
    wo12 = w_o[0:2 * P, :]

    t1 = min(64, N)
    nt1 = N // t1
    h_l, s_l, xo = pl.pallas_call(
        _proj_kernel,
        out_shape=(jax.ShapeDtypeStruct((N, P), jnp.float32),
                   jax.ShapeDtypeStruct((N, S), jnp.float32),
                   jax.ShapeDtypeStruct((N, C_out), jnp.float32)),
        grid=(nt1,),
        in_specs=[pl.BlockSpec((t1, C_in), lambda i: (i, 0)),
                  pl.BlockSpec((C_in, P), lambda i: (0, 0)),
                  pl.BlockSpec((1, P), lambda i: (0, 0)),
                  pl.BlockSpec((C_in, S), lambda i: (0, 0)),
                  pl.BlockSpec((C_in, C_out), lambda i: (0, 0)),
                  pl.BlockSpec((1, C_out), lambda i: (0, 0))],
        out_specs=(pl.BlockSpec((t1, P), lambda i: (i, 0)),
                   pl.BlockSpec((t1, S), lambda i: (i, 0)),
                   pl.BlockSpec((t1, C_out), lambda i: (i, 0))),
        compiler_params=pltpu.CompilerParams(
            dimension_semantics=("parallel",)),
    )(x, w_h, bh2, w_s, wo3, bo2)

    an = pl.pallas_call(
        _an_kernel,
        out_shape=jax.ShapeDtypeStruct((1, N), jnp.float32),
        grid=(1,),
        in_specs=[pl.BlockSpec((N, S), lambda i: (0, 0))],
        out_specs=pl.BlockSpec((1, N), lambda i: (0, 0)),
        compiler_params=pltpu.CompilerParams(
            dimension_semantics=("arbitrary",),
            vmem_limit_bytes=64 * 1024 * 1024),
    )(s_l)

    t2 = min(128, N)
    nt2 = N // t2
    nbr, wgt = pl.pallas_call(
        functools.partial(_knn_kernel, t2, N),
        out_shape=(jax.ShapeDtypeStruct((N, _K), jnp.int32),
                   jax.ShapeDtypeStruct((N, _K), jnp.float32)),
        grid=(nt2,),
        in_specs=[pl.BlockSpec((t2, S), lambda i: (i, 0)),
                  pl.BlockSpec((N, S), lambda i: (0, 0)),
                  pl.BlockSpec((1, N), lambda i: (0, 0))],
        out_specs=(pl.BlockSpec((t2, _K), lambda i: (i, 0)),
                   pl.BlockSpec((t2, _K), lambda i: (i, 0))),
        compiler_params=pltpu.CompilerParams(
            dimension_semantics=("parallel",),
            vmem_limit_bytes=64 * 1024 * 1024),
    )(s_l, s_l, an)

    t3 = min(128, N)
    nt3 = N // t3
    h3 = h_l.reshape(N, 1, P)
    nbr_flat = nbr.reshape(N * _K)
    out = pl.pallas_call(
        functools.partial(_agg_kernel, t3),
        out_shape=jax.ShapeDtypeStruct((N, C_out), jnp.float32),
        grid_spec=pltpu.PrefetchScalarGridSpec(
            num_scalar_prefetch=1,
            grid=(nt3,),
            in_specs=[pl.BlockSpec((t3, C_out), lambda i, nb: (i, 0)),
                      pl.BlockSpec((t3, _K), lambda i, nb: (i, 0)),
                      pl.BlockSpec((N, 1, P), lambda i, nb: (0, 0, 0)),
                      pl.BlockSpec((2 * P, C_out), lambda i, nb: (0, 0))],
            out_specs=pl.BlockSpec((t3, C_out), lambda i, nb: (i, 0)),
            scratch_shapes=[pltpu.VMEM((t3, P), jnp.float32)
                            for _ in range(_K)],
        ),
        compiler_params=pltpu.CompilerParams(
            dimension_semantics=("arbitrary",)),
    )(nbr_flat, xo, wgt, h3, wo12)

    return out, nbr, s_l


# speedup vs baseline: 2.2937x; 1.1374x over previous
import functools

import jax
import jax.numpy as jnp
from jax import lax
from jax.experimental import pallas as pl
from jax.experimental.pallas import tpu as pltpu

_K = 4
_BIG = 1e30


def _proj_kernel(st, x_ref, wh_ref, bh_ref, ws_ref, wo3_ref, bo_ref,
                 h_ref, s_ref, xo_ref):
    xv = x_ref[...]
    h_ref[...] = (jnp.dot(xv, wh_ref[...], preferred_element_type=jnp.float32)
                  + bh_ref[...])
    ws = ws_ref[...]
    for i in range(x_ref.shape[0] // st):
        s_ref[i * st:(i + 1) * st, :] = jnp.dot(
            xv[i * st:(i + 1) * st, :], ws,
            preferred_element_type=jnp.float32)
    xo_ref[...] = (jnp.dot(xv, wo3_ref[...], preferred_element_type=jnp.float32)
                   + bo_ref[...])


def _an_kernel(s_all_ref, an_ref):
    s_all = s_all_ref[...]
    an_ref[...] = lax.dot_general(
        jnp.ones((1, s_all.shape[1]), jnp.float32), s_all * s_all,
        dimension_numbers=(((1,), (1,)), ((), ())),
        preferred_element_type=jnp.float32)


def _knn_kernel(tile, n_total, sq_ref, s_all_ref, an_ref, nbr_ref, w_ref):
    t0 = pl.program_id(0) * tile
    T = tile
    N = s_all_ref.shape[0]
    NB = N // 128

    s_q = sq_ref[...]
    qn = jnp.sum(s_q * s_q, axis=-1, keepdims=True)
    anv = an_ref[...]

    big = jnp.float32(_BIG)
    rowf = (t0 + lax.broadcasted_iota(jnp.int32, (T, 1), 0)
            ).astype(jnp.float32)
    lane = lax.broadcasted_iota(jnp.int32, (T, 128), 1).astype(
        jnp.float32)

    r1 = jnp.full((T, 128), big, jnp.float32)
    r2 = r1
    r3 = r1
    r4 = r1
    b1 = jnp.zeros((T, 128), jnp.float32)
    b2 = b1
    b3 = b1
    b4 = b1
    CB = 16
    for blk in range(NB):
        if blk % CB == 0:
            ch = blk // CB
            cross_c = lax.dot_general(
                s_q, s_all_ref[ch * CB * 128:(ch + 1) * CB * 128, :],
                dimension_numbers=(((1,), (1,)), ((), ())),
                preferred_element_type=jnp.float32)
        sl = slice(blk * 128, (blk + 1) * 128)
        lsl = slice((blk % CB) * 128, (blk % CB + 1) * 128)
        v = (qn + anv[:, sl]) - 2.0 * cross_c[:, lsl]
        colb = lane + jnp.float32(blk * 128)
        v = jnp.where(colb == rowf, big, v)
        fb = jnp.float32(blk)
        c1 = v < r1
        c2 = v < r2
        c3 = v < r3
        c4 = v < r4
        r4 = jnp.where(c3, r3, jnp.where(c4, v, r4))
        b4 = jnp.where(c3, b3, jnp.where(c4, fb, b4))
        r3 = jnp.where(c2, r2, jnp.where(c3, v, r3))
        b3 = jnp.where(c2, b2, jnp.where(c3, fb, b3))
        r2 = jnp.where(c1, r1, jnp.where(c2, v, r2))
        b2 = jnp.where(c1, b1, jnp.where(c2, fb, b2))
        r1 = jnp.where(c1, v, r1)
        b1 = jnp.where(c1, fb, b1)

    V = jnp.concatenate([r1, r2, r3, r4], axis=-1)
    lane4 = jnp.concatenate([lane, lane, lane, lane], axis=-1)
    C = (jnp.concatenate([b1, b2, b3, b4], axis=-1) * jnp.float32(128.0)
         + lane4)

    nf = jnp.float32(n_total)
    idx_cols = []
    w_cols = []
    for rr in range(_K):
        m = jnp.min(V, axis=-1, keepdims=True)
        cand = jnp.where(V <= m, C, nf)
        j = jnp.min(cand, axis=-1, keepdims=True)
        idx_cols.append(j)
        w_cols.append(jnp.exp(-m))
        if rr < _K - 1:
            V = jnp.where(cand == j, big, V)
    nbr_ref[...] = jnp.concatenate(idx_cols, axis=-1).astype(jnp.int32)
    w_ref[...] = jnp.concatenate(w_cols, axis=-1)


def _agg_kernel(tile, nbr_smem, xo_ref, w_ref, h3_ref, wo12_ref, out_ref,
                g0, g1, g2, g3):
    P = h3_ref.shape[2]
    base = pl.program_id(0) * (tile * _K)
    for mi in range(tile):
        o = base + mi * _K
        g0[mi] = h3_ref[nbr_smem[o], 0]
        g1[mi] = h3_ref[nbr_smem[o + 1], 0]
        g2[mi] = h3_ref[nbr_smem[o + 2], 0]
        g3[mi] = h3_ref[nbr_smem[o + 3], 0]
    wv = w_ref[...]
    m0 = g0[...] * wv[:, 0:1]
    m1 = g1[...] * wv[:, 1:2]
    m2 = g2[...] * wv[:, 2:3]
    m3 = g3[...] * wv[:, 3:4]
    mean = (((m0 + m1) + m2) + m3) * jnp.float32(1.0 / _K)
    mx = jnp.maximum(jnp.maximum(jnp.maximum(m0, m1), m2), m3)
    out_ref[...] = (jnp.dot(mean, wo12_ref[0:P, :],
                            preferred_element_type=jnp.float32)
                    + jnp.dot(mx, wo12_ref[P:2 * P, :],
                              preferred_element_type=jnp.float32)
                    + xo_ref[...])


def kernel(x, w_h, b_h, w_s, w_o, b_o):
    N, C_in = x.shape
    P = w_h.shape[1]
    S = w_s.shape[1]
    C_out = w_o.shape[1]

    x = x.astype(jnp.float32)
    w_h = w_h.astype(jnp.float32)
    w_s = w_s.astype(jnp.float32)
    w_o = w_o.astype(jnp.float32)
    bh2 = b_h.reshape(1, P).astype(jnp.float32)
    bo2 = b_o.reshape(1, C_out).astype(jnp.float32)
    wo3 = w_o[2 * P:2 * P + C_in, :]
    wo12 = w_o[0:2 * P, :]

    t1 = min(2048, N)
    nt1 = N // t1
    h_l, s_l, xo = pl.pallas_call(
        functools.partial(_proj_kernel, min(64, N)),
        out_shape=(jax.ShapeDtypeStruct((N, P), jnp.float32),
                   jax.ShapeDtypeStruct((N, S), jnp.float32),
                   jax.ShapeDtypeStruct((N, C_out), jnp.float32)),
        grid=(nt1,),
        in_specs=[pl.BlockSpec((t1, C_in), lambda i: (i, 0)),
                  pl.BlockSpec((C_in, P), lambda i: (0, 0)),
                  pl.BlockSpec((1, P), lambda i: (0, 0)),
                  pl.BlockSpec((C_in, S), lambda i: (0, 0)),
                  pl.BlockSpec((C_in, C_out), lambda i: (0, 0)),
                  pl.BlockSpec((1, C_out), lambda i: (0, 0))],
        out_specs=(pl.BlockSpec((t1, P), lambda i: (i, 0)),
                   pl.BlockSpec((t1, S), lambda i: (i, 0)),
                   pl.BlockSpec((t1, C_out), lambda i: (i, 0))),
        compiler_params=pltpu.CompilerParams(
            dimension_semantics=("parallel",)),
    )(x, w_h, bh2, w_s, wo3, bo2)

    an = pl.pallas_call(
        _an_kernel,
        out_shape=jax.ShapeDtypeStruct((1, N), jnp.float32),
        grid=(1,),
        in_specs=[pl.BlockSpec((N, S), lambda i: (0, 0))],
        out_specs=pl.BlockSpec((1, N), lambda i: (0, 0)),
        compiler_params=pltpu.CompilerParams(
            dimension_semantics=("arbitrary",),
            vmem_limit_bytes=64 * 1024 * 1024),
    )(s_l)

    t2 = min(128, N)
    nt2 = N // t2
    nbr, wgt = pl.pallas_call(
        functools.partial(_knn_kernel, t2, N),
        out_shape=(jax.ShapeDtypeStruct((N, _K), jnp.int32),
                   jax.ShapeDtypeStruct((N, _K), jnp.float32)),
        grid=(nt2,),
        in_specs=[pl.BlockSpec((t2, S), lambda i: (i, 0)),
                  pl.BlockSpec((N, S), lambda i: (0, 0)),
                  pl.BlockSpec((1, N), lambda i: (0, 0))],
        out_specs=(pl.BlockSpec((t2, _K), lambda i: (i, 0)),
                   pl.BlockSpec((t2, _K), lambda i: (i, 0))),
        compiler_params=pltpu.CompilerParams(
            dimension_semantics=("parallel",),
            vmem_limit_bytes=64 * 1024 * 1024),
    )(s_l, s_l, an)

    t3 = min(256, N)
    nt3 = N // t3
    h3 = h_l.reshape(N, 1, P)
    nbr_flat = nbr.reshape(N * _K)
    out = pl.pallas_call(
        functools.partial(_agg_kernel, t3),
        out_shape=jax.ShapeDtypeStruct((N, C_out), jnp.float32),
        grid_spec=pltpu.PrefetchScalarGridSpec(
            num_scalar_prefetch=1,
            grid=(nt3,),
            in_specs=[pl.BlockSpec((t3, C_out), lambda i, nb: (i, 0)),
                      pl.BlockSpec((t3, _K), lambda i, nb: (i, 0)),
                      pl.BlockSpec((N, 1, P), lambda i, nb: (0, 0, 0)),
                      pl.BlockSpec((2 * P, C_out), lambda i, nb: (0, 0))],
            out_specs=pl.BlockSpec((t3, C_out), lambda i, nb: (i, 0)),
            scratch_shapes=[pltpu.VMEM((t3, P), jnp.float32)
                            for _ in range(_K)],
        ),
        compiler_params=pltpu.CompilerParams(
            dimension_semantics=("arbitrary",)),
    )(nbr_flat, xo, wgt, h3, wo12)

    return out, nbr, s_l


# speedup vs baseline: 2.4901x; 1.0856x over previous
import functools

import jax
import jax.numpy as jnp
from jax import lax
from jax.experimental import pallas as pl
from jax.experimental.pallas import tpu as pltpu

_K = 4
_BIG = 1e30


def _proj_kernel(st, x_ref, wh_ref, bh_ref, ws_ref, wo3_ref, bo_ref,
                 h_ref, s_ref, xo_ref):
    xv = x_ref[...]
    h_ref[...] = (jnp.dot(xv, wh_ref[...], preferred_element_type=jnp.float32)
                  + bh_ref[...])
    ws = ws_ref[...]
    for i in range(x_ref.shape[0] // st):
        s_ref[i * st:(i + 1) * st, :] = jnp.dot(
            xv[i * st:(i + 1) * st, :], ws,
            preferred_element_type=jnp.float32)
    xo_ref[...] = (jnp.dot(xv, wo3_ref[...], preferred_element_type=jnp.float32)
                   + bo_ref[...])


def _an_kernel(s_all_ref, an_ref):
    s_all = s_all_ref[...]
    an_ref[...] = lax.dot_general(
        jnp.ones((1, s_all.shape[1]), jnp.float32), s_all * s_all,
        dimension_numbers=(((1,), (1,)), ((), ())),
        preferred_element_type=jnp.float32)


def _knn_kernel(tile, n_total, sq_ref, s_all_ref, an_ref, nbr_ref, w_ref):
    t0 = pl.program_id(0) * tile
    T = tile
    N = s_all_ref.shape[0]
    NB = N // 128

    s_q = sq_ref[...]
    qn = jnp.sum(s_q * s_q, axis=-1, keepdims=True)
    anv = an_ref[...]

    big = jnp.float32(_BIG)
    rowf = (t0 + lax.broadcasted_iota(jnp.int32, (T, 1), 0)
            ).astype(jnp.float32)
    lane = lax.broadcasted_iota(jnp.int32, (T, 128), 1).astype(
        jnp.float32)

    r1 = jnp.full((T, 128), big, jnp.float32)
    r2 = r1
    r3 = r1
    r4 = r1
    b1 = jnp.zeros((T, 128), jnp.float32)
    b2 = b1
    b3 = b1
    b4 = b1
    CB = 16
    for blk in range(NB):
        if blk % CB == 0:
            ch = blk // CB
            cross_c = lax.dot_general(
                s_q, s_all_ref[ch * CB * 128:(ch + 1) * CB * 128, :],
                dimension_numbers=(((1,), (1,)), ((), ())),
                preferred_element_type=jnp.float32)
        sl = slice(blk * 128, (blk + 1) * 128)
        lsl = slice((blk % CB) * 128, (blk % CB + 1) * 128)
        v = (qn + anv[:, sl]) - 2.0 * cross_c[:, lsl]
        colb = lane + jnp.float32(blk * 128)
        v = jnp.where(colb == rowf, big, v)
        fb = jnp.float32(blk)
        c1 = v < r1
        c2 = v < r2
        c3 = v < r3
        c4 = v < r4
        r4 = jnp.where(c3, r3, jnp.where(c4, v, r4))
        b4 = jnp.where(c3, b3, jnp.where(c4, fb, b4))
        r3 = jnp.where(c2, r2, jnp.where(c3, v, r3))
        b3 = jnp.where(c2, b2, jnp.where(c3, fb, b3))
        r2 = jnp.where(c1, r1, jnp.where(c2, v, r2))
        b2 = jnp.where(c1, b1, jnp.where(c2, fb, b2))
        r1 = jnp.where(c1, v, r1)
        b1 = jnp.where(c1, fb, b1)

    V = jnp.concatenate([r1, r2, r3, r4], axis=-1)
    lane4 = jnp.concatenate([lane, lane, lane, lane], axis=-1)
    C = (jnp.concatenate([b1, b2, b3, b4], axis=-1) * jnp.float32(128.0)
         + lane4)

    nf = jnp.float32(n_total)
    idx_cols = []
    w_cols = []
    for rr in range(_K):
        m = jnp.min(V, axis=-1, keepdims=True)
        cand = jnp.where(V <= m, C, nf)
        j = jnp.min(cand, axis=-1, keepdims=True)
        idx_cols.append(j)
        w_cols.append(jnp.exp(-m))
        if rr < _K - 1:
            V = jnp.where(cand == j, big, V)
    nbr_ref[...] = jnp.concatenate(idx_cols, axis=-1).astype(jnp.int32)
    w_ref[...] = jnp.concatenate(w_cols, axis=-1)


def _agg_kernel(tile, nbr_smem, xo_ref, w_ref, h3_ref, wo12_ref, out_ref,
                g0, g1, g2, g3):
    P = h3_ref.shape[2]
    base = pl.program_id(0) * (tile * _K)
    for mi in range(tile):
        o = base + mi * _K
        g0[mi] = h3_ref[nbr_smem[o], 0]
        g1[mi] = h3_ref[nbr_smem[o + 1], 0]
        g2[mi] = h3_ref[nbr_smem[o + 2], 0]
        g3[mi] = h3_ref[nbr_smem[o + 3], 0]
    wv = w_ref[...]
    m0 = g0[...] * wv[:, 0:1]
    m1 = g1[...] * wv[:, 1:2]
    m2 = g2[...] * wv[:, 2:3]
    m3 = g3[...] * wv[:, 3:4]
    mean = (((m0 + m1) + m2) + m3) * jnp.float32(1.0 / _K)
    mx = jnp.maximum(jnp.maximum(jnp.maximum(m0, m1), m2), m3)
    out_ref[...] = (jnp.dot(mean, wo12_ref[0:P, :],
                            preferred_element_type=jnp.float32)
                    + jnp.dot(mx, wo12_ref[P:2 * P, :],
                              preferred_element_type=jnp.float32)
                    + xo_ref[...])


def kernel(x, w_h, b_h, w_s, w_o, b_o):
    N, C_in = x.shape
    P = w_h.shape[1]
    S = w_s.shape[1]
    C_out = w_o.shape[1]

    x = x.astype(jnp.float32)
    w_h = w_h.astype(jnp.float32)
    w_s = w_s.astype(jnp.float32)
    w_o = w_o.astype(jnp.float32)
    bh2 = b_h.reshape(1, P).astype(jnp.float32)
    bo2 = b_o.reshape(1, C_out).astype(jnp.float32)
    wo3 = w_o[2 * P:2 * P + C_in, :]
    wo12 = w_o[0:2 * P, :]

    t1 = min(2048, N)
    nt1 = N // t1
    h_l, s_l, xo = pl.pallas_call(
        functools.partial(_proj_kernel, min(64, N)),
        out_shape=(jax.ShapeDtypeStruct((N, P), jnp.float32),
                   jax.ShapeDtypeStruct((N, S), jnp.float32),
                   jax.ShapeDtypeStruct((N, C_out), jnp.float32)),
        grid=(nt1,),
        in_specs=[pl.BlockSpec((t1, C_in), lambda i: (i, 0)),
                  pl.BlockSpec((C_in, P), lambda i: (0, 0)),
                  pl.BlockSpec((1, P), lambda i: (0, 0)),
                  pl.BlockSpec((C_in, S), lambda i: (0, 0)),
                  pl.BlockSpec((C_in, C_out), lambda i: (0, 0)),
                  pl.BlockSpec((1, C_out), lambda i: (0, 0))],
        out_specs=(pl.BlockSpec((t1, P), lambda i: (i, 0)),
                   pl.BlockSpec((t1, S), lambda i: (i, 0)),
                   pl.BlockSpec((t1, C_out), lambda i: (i, 0))),
        compiler_params=pltpu.CompilerParams(
            dimension_semantics=("parallel",)),
    )(x, w_h, bh2, w_s, wo3, bo2)

    an = pl.pallas_call(
        _an_kernel,
        out_shape=jax.ShapeDtypeStruct((1, N), jnp.float32),
        grid=(1,),
        in_specs=[pl.BlockSpec((N, S), lambda i: (0, 0))],
        out_specs=pl.BlockSpec((1, N), lambda i: (0, 0)),
        compiler_params=pltpu.CompilerParams(
            dimension_semantics=("arbitrary",),
            vmem_limit_bytes=64 * 1024 * 1024),
    )(s_l)

    t2 = min(256, N)
    nt2 = N // t2
    nbr, wgt = pl.pallas_call(
        functools.partial(_knn_kernel, t2, N),
        out_shape=(jax.ShapeDtypeStruct((N, _K), jnp.int32),
                   jax.ShapeDtypeStruct((N, _K), jnp.float32)),
        grid=(nt2,),
        in_specs=[pl.BlockSpec((t2, S), lambda i: (i, 0)),
                  pl.BlockSpec((N, S), lambda i: (0, 0)),
                  pl.BlockSpec((1, N), lambda i: (0, 0))],
        out_specs=(pl.BlockSpec((t2, _K), lambda i: (i, 0)),
                   pl.BlockSpec((t2, _K), lambda i: (i, 0))),
        compiler_params=pltpu.CompilerParams(
            dimension_semantics=("parallel",),
            vmem_limit_bytes=64 * 1024 * 1024),
    )(s_l, s_l, an)

    t3 = min(256, N)
    nt3 = N // t3
    h3 = h_l.reshape(N, 1, P)
    nbr_flat = nbr.reshape(N * _K)
    out = pl.pallas_call(
        functools.partial(_agg_kernel, t3),
        out_shape=jax.ShapeDtypeStruct((N, C_out), jnp.float32),
        grid_spec=pltpu.PrefetchScalarGridSpec(
            num_scalar_prefetch=1,
            grid=(nt3,),
            in_specs=[pl.BlockSpec((t3, C_out), lambda i, nb: (i, 0)),
                      pl.BlockSpec((t3, _K), lambda i, nb: (i, 0)),
                      pl.BlockSpec((N, 1, P), lambda i, nb: (0, 0, 0)),
                      pl.BlockSpec((2 * P, C_out), lambda i, nb: (0, 0))],
            out_specs=pl.BlockSpec((t3, C_out), lambda i, nb: (i, 0)),
            scratch_shapes=[pltpu.VMEM((t3, P), jnp.float32)
                            for _ in range(_K)],
        ),
        compiler_params=pltpu.CompilerParams(
            dimension_semantics=("arbitrary",)),
    )(nbr_flat, xo, wgt, h3, wo12)

    return out, nbr, s_l


# speedup vs baseline: 2.5466x; 1.0227x over previous
import functools

import jax
import jax.numpy as jnp
from jax import lax
from jax.experimental import pallas as pl
from jax.experimental.pallas import tpu as pltpu

_K = 4
_BIG = 1e30


def _proj_kernel(st, x_ref, wh_ref, bh_ref, ws_ref, wo3_ref, bo_ref,
                 h_ref, s_ref, xo_ref, an_ref):
    xv = x_ref[...]
    h_ref[...] = (jnp.dot(xv, wh_ref[...], preferred_element_type=jnp.float32)
                  + bh_ref[...])
    ws = ws_ref[...]
    for i in range(x_ref.shape[0] // st):
        s_ref[i * st:(i + 1) * st, :] = jnp.dot(
            xv[i * st:(i + 1) * st, :], ws,
            preferred_element_type=jnp.float32)
    xo_ref[...] = (jnp.dot(xv, wo3_ref[...], preferred_element_type=jnp.float32)
                   + bo_ref[...])
    sv = s_ref[...]
    an_ref[...] = lax.dot_general(
        jnp.ones((1, sv.shape[1]), jnp.float32), sv * sv,
        dimension_numbers=(((1,), (1,)), ((), ())),
        preferred_element_type=jnp.float32)


def _knn_kernel(tile, n_total, sq_ref, s_all_ref, an_ref, nbr_ref, w_ref):
    t0 = pl.program_id(0) * tile
    T = tile
    N = s_all_ref.shape[0]
    NB = N // 128

    s_q = sq_ref[...]
    qn = jnp.sum(s_q * s_q, axis=-1, keepdims=True)
    anv = an_ref[...]

    big = jnp.float32(_BIG)
    rowf = (t0 + lax.broadcasted_iota(jnp.int32, (T, 1), 0)
            ).astype(jnp.float32)
    lane = lax.broadcasted_iota(jnp.int32, (T, 128), 1).astype(
        jnp.float32)

    r1 = jnp.full((T, 128), big, jnp.float32)
    r2 = r1
    r3 = r1
    r4 = r1
    b1 = jnp.zeros((T, 128), jnp.float32)
    b2 = b1
    b3 = b1
    b4 = b1
    CB = 16
    for blk in range(NB):
        if blk % CB == 0:
            ch = blk // CB
            cross_c = lax.dot_general(
                s_q, s_all_ref[ch * CB * 128:(ch + 1) * CB * 128, :],
                dimension_numbers=(((1,), (1,)), ((), ())),
                preferred_element_type=jnp.float32)
        sl = slice(blk * 128, (blk + 1) * 128)
        lsl = slice((blk % CB) * 128, (blk % CB + 1) * 128)
        v = (qn + anv[:, sl]) - 2.0 * cross_c[:, lsl]
        colb = lane + jnp.float32(blk * 128)
        v = jnp.where(colb == rowf, big, v)
        fb = jnp.float32(blk)
        c1 = v < r1
        c2 = v < r2
        c3 = v < r3
        c4 = v < r4
        r4 = jnp.where(c3, r3, jnp.where(c4, v, r4))
        b4 = jnp.where(c3, b3, jnp.where(c4, fb, b4))
        r3 = jnp.where(c2, r2, jnp.where(c3, v, r3))
        b3 = jnp.where(c2, b2, jnp.where(c3, fb, b3))
        r2 = jnp.where(c1, r1, jnp.where(c2, v, r2))
        b2 = jnp.where(c1, b1, jnp.where(c2, fb, b2))
        r1 = jnp.where(c1, v, r1)
        b1 = jnp.where(c1, fb, b1)

    V = jnp.concatenate([r1, r2, r3, r4], axis=-1)
    lane4 = jnp.concatenate([lane, lane, lane, lane], axis=-1)
    C = (jnp.concatenate([b1, b2, b3, b4], axis=-1) * jnp.float32(128.0)
         + lane4)

    nf = jnp.float32(n_total)
    idx_cols = []
    w_cols = []
    for rr in range(_K):
        m = jnp.min(V, axis=-1, keepdims=True)
        cand = jnp.where(V <= m, C, nf)
        j = jnp.min(cand, axis=-1, keepdims=True)
        idx_cols.append(j)
        w_cols.append(jnp.exp(-m))
        if rr < _K - 1:
            V = jnp.where(cand == j, big, V)
    nbr_ref[...] = jnp.concatenate(idx_cols, axis=-1).astype(jnp.int32)
    w_ref[...] = jnp.concatenate(w_cols, axis=-1)


def _agg_kernel(tile, nbr_smem, xo_ref, w_ref, h3_ref, wo12_ref, out_ref,
                g0, g1, g2, g3):
    P = h3_ref.shape[2]
    base = pl.program_id(0) * (tile * _K)
    for mi in range(tile):
        o = base + mi * _K
        g0[mi] = h3_ref[nbr_smem[o], 0]
        g1[mi] = h3_ref[nbr_smem[o + 1], 0]
        g2[mi] = h3_ref[nbr_smem[o + 2], 0]
        g3[mi] = h3_ref[nbr_smem[o + 3], 0]
    wv = w_ref[...]
    m0 = g0[...] * wv[:, 0:1]
    m1 = g1[...] * wv[:, 1:2]
    m2 = g2[...] * wv[:, 2:3]
    m3 = g3[...] * wv[:, 3:4]
    mean = (((m0 + m1) + m2) + m3) * jnp.float32(1.0 / _K)
    mx = jnp.maximum(jnp.maximum(jnp.maximum(m0, m1), m2), m3)
    out_ref[...] = (jnp.dot(mean, wo12_ref[0:P, :],
                            preferred_element_type=jnp.float32)
                    + jnp.dot(mx, wo12_ref[P:2 * P, :],
                              preferred_element_type=jnp.float32)
                    + xo_ref[...])


def kernel(x, w_h, b_h, w_s, w_o, b_o):
    N, C_in = x.shape
    P = w_h.shape[1]
    S = w_s.shape[1]
    C_out = w_o.shape[1]

    x = x.astype(jnp.float32)
    w_h = w_h.astype(jnp.float32)
    w_s = w_s.astype(jnp.float32)
    w_o = w_o.astype(jnp.float32)
    bh2 = b_h.reshape(1, P).astype(jnp.float32)
    bo2 = b_o.reshape(1, C_out).astype(jnp.float32)
    wo3 = w_o[2 * P:2 * P + C_in, :]
    wo12 = w_o[0:2 * P, :]

    t1 = min(2048, N)
    nt1 = N // t1
    h_l, s_l, xo, an = pl.pallas_call(
        functools.partial(_proj_kernel, min(64, N)),
        out_shape=(jax.ShapeDtypeStruct((N, P), jnp.float32),
                   jax.ShapeDtypeStruct((N, S), jnp.float32),
                   jax.ShapeDtypeStruct((N, C_out), jnp.float32),
                   jax.ShapeDtypeStruct((1, N), jnp.float32)),
        grid=(nt1,),
        in_specs=[pl.BlockSpec((t1, C_in), lambda i: (i, 0)),
                  pl.BlockSpec((C_in, P), lambda i: (0, 0)),
                  pl.BlockSpec((1, P), lambda i: (0, 0)),
                  pl.BlockSpec((C_in, S), lambda i: (0, 0)),
                  pl.BlockSpec((C_in, C_out), lambda i: (0, 0)),
                  pl.BlockSpec((1, C_out), lambda i: (0, 0))],
        out_specs=(pl.BlockSpec((t1, P), lambda i: (i, 0)),
                   pl.BlockSpec((t1, S), lambda i: (i, 0)),
                   pl.BlockSpec((t1, C_out), lambda i: (i, 0)),
                   pl.BlockSpec((1, t1), lambda i: (0, i))),
        compiler_params=pltpu.CompilerParams(
            dimension_semantics=("parallel",)),
    )(x, w_h, bh2, w_s, wo3, bo2)

    t2 = min(256, N)
    nt2 = N // t2
    nbr, wgt = pl.pallas_call(
        functools.partial(_knn_kernel, t2, N),
        out_shape=(jax.ShapeDtypeStruct((N, _K), jnp.int32),
                   jax.ShapeDtypeStruct((N, _K), jnp.float32)),
        grid=(nt2,),
        in_specs=[pl.BlockSpec((t2, S), lambda i: (i, 0)),
                  pl.BlockSpec((N, S), lambda i: (0, 0)),
                  pl.BlockSpec((1, N), lambda i: (0, 0))],
        out_specs=(pl.BlockSpec((t2, _K), lambda i: (i, 0)),
                   pl.BlockSpec((t2, _K), lambda i: (i, 0))),
        compiler_params=pltpu.CompilerParams(
            dimension_semantics=("parallel",),
            vmem_limit_bytes=64 * 1024 * 1024),
    )(s_l, s_l, an)

    t3 = min(512, N)
    nt3 = N // t3
    h3 = h_l.reshape(N, 1, P)
    nbr_flat = nbr.reshape(N * _K)
    out = pl.pallas_call(
        functools.partial(_agg_kernel, t3),
        out_shape=jax.ShapeDtypeStruct((N, C_out), jnp.float32),
        grid_spec=pltpu.PrefetchScalarGridSpec(
            num_scalar_prefetch=1,
            grid=(nt3,),
            in_specs=[pl.BlockSpec((t3, C_out), lambda i, nb: (i, 0)),
                      pl.BlockSpec((t3, _K), lambda i, nb: (i, 0)),
                      pl.BlockSpec((N, 1, P), lambda i, nb: (0, 0, 0)),
                      pl.BlockSpec((2 * P, C_out), lambda i, nb: (0, 0))],
            out_specs=pl.BlockSpec((t3, C_out), lambda i, nb: (i, 0)),
            scratch_shapes=[pltpu.VMEM((t3, P), jnp.float32)
                            for _ in range(_K)],
        ),
        compiler_params=pltpu.CompilerParams(
            dimension_semantics=("arbitrary",)),
    )(nbr_flat, xo, wgt, h3, wo12)

    return out, nbr, s_l
